# baseline (speedup 1.0000x reference)
import jax
import jax.numpy as jnp
from jax import lax
from jax.experimental import pallas as pl
from jax.experimental.pallas import tpu as pltpu
from jax.experimental.pallas import tpu_sc as plsc

B, L, V, D = 4096, 200, 100000, 128
DE = 8
DO = D + 2 * DE
N_TOK = B * L

NC, NS = 2, 16
NW = NC * NS
TOK_PER_W = N_TOK // NW
K = 2
CHUNK = K * 128
N_CHUNKS = TOK_PER_W // CHUNK
ROWS_PER_W = TOK_PER_W // 128


def _body(x_hbm, xe_hbm, xn_hbm, w_hbm, wen_hbm, out_hbm,
          idx0, e0, n0, word0, en0,
          idx1, e1, n1, word1, en1,
          wen_v,
          sem_g, sem_out0, sem_out1, sem_idx0, sem_idx1):
    wid = lax.axis_index("s") * NC + lax.axis_index("c")
    tok0 = wid * TOK_PER_W
    row0 = wid * ROWS_PER_W

    slots = ((idx0, e0, n0, word0, en0, sem_out0, sem_idx0),
             (idx1, e1, n1, word1, en1, sem_out1, sem_idx1))

    pltpu.sync_copy(wen_hbm, wen_v)

    def issue_idx_loads(c, slot):
        idx_v, e_v, n_v, sem_idx = slot[0], slot[1], slot[2], slot[6]
        r = row0 + c * K
        pltpu.async_copy(x_hbm.at[pl.ds(r, K)], idx_v, sem_idx)
        pltpu.async_copy(xe_hbm.at[pl.ds(r, K)], e_v, sem_idx)
        pltpu.async_copy(xn_hbm.at[pl.ds(r, K)], n_v, sem_idx)

    def wait_idx_loads(slot):
        idx_v, e_v, n_v, sem_idx = slot[0], slot[1], slot[2], slot[6]
        pltpu.make_async_copy(x_hbm.at[pl.ds(row0, K)], idx_v, sem_idx).wait()
        pltpu.make_async_copy(xe_hbm.at[pl.ds(row0, K)], e_v, sem_idx).wait()
        pltpu.make_async_copy(xn_hbm.at[pl.ds(row0, K)], n_v, sem_idx).wait()

    def out_slices(c):
        base = tok0 + c * CHUNK
        return (out_hbm.at[pl.ds(base, CHUNK), pl.ds(0, D)],
                out_hbm.at[pl.ds(base, CHUNK), pl.ds(D, 2 * DE)])

    iota16 = lax.iota(jnp.int32, 16)

    def do_chunk(t, s):
        slot = slots[s]
        other = slots[1 - s]
        idx_v, e_v, n_v, word_v, en_v, sem_out, _ = slot
        c = t * 2 + s
        wait_idx_loads(slot)

        @pl.when(c < N_CHUNKS - 1)
        def _():
            issue_idx_loads(c + 1, other)

        @pl.when(t >= 1)
        def _():
            wdst, edst = out_slices(c)
            pltpu.make_async_copy(word_v, wdst, sem_out).wait()
            pltpu.make_async_copy(en_v, edst, sem_out).wait()

        cps = [pltpu.async_copy(
            w_hbm.at[idx_v.at[j]], word_v.at[pl.ds(j * 128, 128)], sem_g)
            for j in range(K)]

        for g in range(CHUNK // 16):
            j, u = g // 8, g % 8
            sl = pl.ds(u * 16, 16)
            combo16 = (e_v[j, sl] * 3 + n_v[j, sl]) * 16
            toks = g * 16 + iota16
            for f in range(16):
                vals = plsc.load_gather(wen_v, [combo16 + f])
                plsc.store_scatter(en_v, [toks, jnp.full((16,), f, jnp.int32)],
                                   vals)

        for cp in cps:
            cp.wait()
        wdst, edst = out_slices(c)
        pltpu.async_copy(word_v, wdst, sem_out)
        pltpu.async_copy(en_v, edst, sem_out)

    issue_idx_loads(0, slots[0])

    def outer(t, carry):
        do_chunk(t, 0)
        do_chunk(t, 1)
        return carry

    lax.fori_loop(0, N_CHUNKS // 2, outer, 0)

    for s in range(2):
        slot = slots[s]
        wdst, edst = out_slices(N_CHUNKS - 2 + s)
        pltpu.make_async_copy(slot[3], wdst, slot[5]).wait()
        pltpu.make_async_copy(slot[4], edst, slot[5]).wait()


@jax.jit
def _run(x2d, xe2d, xn2d, w, w_en):
    mesh = plsc.VectorSubcoreMesh(core_axis_name="c", subcore_axis_name="s")
    slot_scratch = [
        pltpu.VMEM((K, 128), jnp.int32),
        pltpu.VMEM((K, 128), jnp.int32),
        pltpu.VMEM((K, 128), jnp.int32),
        pltpu.VMEM((CHUNK, D), jnp.float32),
        pltpu.VMEM((CHUNK, 2 * DE), jnp.float32),
    ]
    f = pl.kernel(
        _body,
        out_type=jax.ShapeDtypeStruct((N_TOK, DO), jnp.float32),
        mesh=mesh,
        scratch_types=slot_scratch + slot_scratch + [
            pltpu.VMEM((256,), jnp.float32),
            pltpu.SemaphoreType.DMA,
            pltpu.SemaphoreType.DMA,
            pltpu.SemaphoreType.DMA,
            pltpu.SemaphoreType.DMA,
            pltpu.SemaphoreType.DMA,
        ],
        compiler_params=pltpu.CompilerParams(
            use_tc_tiling_on_sc=False, needs_layout_passes=False),
    )
    return f(x2d, xe2d, xn2d, w, w_en)


def kernel(x, x_entity, x_negation, target, text_inputs, use_elmo,
           W, W_entity, W_negation):
    w_en = jnp.zeros((16, 16), jnp.float32)
    w_en = w_en.at[:9, :8].set(jnp.repeat(W_entity, 3, axis=0))
    w_en = w_en.at[:9, 8:].set(jnp.tile(W_negation, (3, 1)))
    shp = (N_TOK // 128, 128)
    out = _run(x.reshape(shp).astype(jnp.int32),
               x_entity.reshape(shp).astype(jnp.int32),
               x_negation.reshape(shp).astype(jnp.int32),
               W, w_en.reshape(256))
    return out.reshape(B, L, DO)


# speedup vs baseline: 1.3721x; 1.3721x over previous
import jax
import jax.numpy as jnp
from jax import lax
from jax.experimental import pallas as pl
from jax.experimental.pallas import tpu as pltpu
from jax.experimental.pallas import tpu_sc as plsc

B, L, V, D = 4096, 200, 100000, 128
DE = 8
DO = D + 2 * DE
N_TOK = B * L

NC, NS = 2, 16
NW = NC * NS
TOK_PER_W = N_TOK // NW
K = 2
CHUNK = K * 128
N_CHUNKS = TOK_PER_W // CHUNK
ROWS_PER_W = TOK_PER_W // 128

N_ROWS = N_TOK // 128
TC_BLK = 8192


def _sc_body(x_hbm, w_hbm, out_hbm,
             idx0, word0, idx1, word1,
             sem_g, sem_out0, sem_out1, sem_idx0, sem_idx1):
    wid = lax.axis_index("s") * NC + lax.axis_index("c")
    tok0 = wid * TOK_PER_W
    row0 = wid * ROWS_PER_W

    slots = ((idx0, word0, sem_out0, sem_idx0),
             (idx1, word1, sem_out1, sem_idx1))

    def out_slice(c):
        base = tok0 + c * CHUNK
        return out_hbm.at[pl.ds(base, CHUNK), pl.ds(0, D)]

    def do_chunk(t, s):
        idx_v, word_v, sem_out, sem_idx = slots[s]
        other_idx, _, _, other_sem = slots[1 - s]
        c = t * 2 + s
        pltpu.make_async_copy(
            x_hbm.at[pl.ds(row0, K)], idx_v, sem_idx).wait()

        @pl.when(c < N_CHUNKS - 1)
        def _():
            r = row0 + (c + 1) * K
            pltpu.async_copy(x_hbm.at[pl.ds(r, K)], other_idx, other_sem)

        @pl.when(t >= 1)
        def _():
            pltpu.make_async_copy(word_v, out_slice(c), sem_out).wait()

        cps = [pltpu.async_copy(
            w_hbm.at[idx_v.at[j]], word_v.at[pl.ds(j * 128, 128)], sem_g)
            for j in range(K)]
        for cp in cps:
            cp.wait()
        pltpu.async_copy(word_v, out_slice(c), sem_out)

    pltpu.async_copy(x_hbm.at[pl.ds(row0, K)], idx0, sem_idx0)

    def outer(t, carry):
        do_chunk(t, 0)
        do_chunk(t, 1)
        return carry

    lax.fori_loop(0, N_CHUNKS // 2, outer, 0)

    for s in range(2):
        idx_v, word_v, sem_out, _ = slots[s]
        pltpu.make_async_copy(word_v, out_slice(N_CHUNKS - 2 + s),
                              sem_out).wait()


def _tc_body(xe_ref, xn_ref, went_ref, wneg_ref, out_ref):
    e = xe_ref[...]
    n = xn_ref[...]
    went = went_ref[...]
    wneg = wneg_ref[...]
    ent = jnp.zeros((DE, TC_BLK), jnp.float32)
    neg = jnp.zeros((DE, TC_BLK), jnp.float32)
    for r in range(3):
        ent = ent + jnp.where(e == r, went[:, r:r + 1], 0.0)
        neg = neg + jnp.where(n == r, wneg[:, r:r + 1], 0.0)
    out_ref[...] = jnp.concatenate([ent, neg], axis=0)


@jax.jit
def _run(x2d, xe2d, xn2d, w, w_ent, w_neg):
    mesh = plsc.VectorSubcoreMesh(core_axis_name="c", subcore_axis_name="s")
    sc = pl.kernel(
        _sc_body,
        out_type=jax.ShapeDtypeStruct((N_TOK, DO), jnp.float32),
        mesh=mesh,
        scratch_types=[
            pltpu.VMEM((K, 128), jnp.int32),
            pltpu.VMEM((CHUNK, D), jnp.float32),
            pltpu.VMEM((K, 128), jnp.int32),
            pltpu.VMEM((CHUNK, D), jnp.float32),
            pltpu.SemaphoreType.DMA,
            pltpu.SemaphoreType.DMA,
            pltpu.SemaphoreType.DMA,
            pltpu.SemaphoreType.DMA,
            pltpu.SemaphoreType.DMA,
        ],
        compiler_params=pltpu.CompilerParams(
            use_tc_tiling_on_sc=True, needs_layout_passes=False),
    )
    out_words = sc(x2d, w)

    grid = (N_TOK // TC_BLK,)
    en_t = pl.pallas_call(
        _tc_body,
        grid=grid,
        in_specs=[
            pl.BlockSpec((1, TC_BLK), lambda i: (0, i)),
            pl.BlockSpec((1, TC_BLK), lambda i: (0, i)),
            pl.BlockSpec((DE, 3), lambda i: (0, 0)),
            pl.BlockSpec((DE, 3), lambda i: (0, 0)),
        ],
        out_specs=pl.BlockSpec((2 * DE, TC_BLK), lambda i: (0, i)),
        out_shape=jax.ShapeDtypeStruct((2 * DE, N_TOK), jnp.float32),
    )(xe2d, xn2d, w_ent, w_neg)
    return out_words.at[:, D:].set(en_t.T)


def kernel(x, x_entity, x_negation, target, text_inputs, use_elmo,
           W, W_entity, W_negation):
    out = _run(x.reshape(N_ROWS, 128).astype(jnp.int32),
               x_entity.reshape(1, N_TOK).astype(jnp.int32),
               x_negation.reshape(1, N_TOK).astype(jnp.int32),
               W, W_entity.T, W_negation.T)
    return out.reshape(B, L, DO)


# speedup vs baseline: 1.4608x; 1.0646x over previous
import jax
import jax.numpy as jnp
from jax import lax
from jax.experimental import pallas as pl
from jax.experimental.pallas import tpu as pltpu
from jax.experimental.pallas import tpu_sc as plsc

B, L, V, D = 4096, 200, 100000, 128
DE = 8
DO = D + 2 * DE
N_TOK = B * L

NC, NS = 2, 16
NW = NC * NS
TOK_PER_W = N_TOK // NW
K = 2
CHUNK = K * 128
N_CHUNKS = TOK_PER_W // CHUNK
ROWS_PER_W = TOK_PER_W // 128

N_ROWS = N_TOK // 128
TC_BLK = 8192


def _sc_body(x_hbm, w_hbm, out_hbm,
             idx0, word0, idx1, word1,
             sem_g, sem_out0, sem_out1, sem_idx0, sem_idx1):
    wid = lax.axis_index("s") * NC + lax.axis_index("c")
    tok0 = wid * TOK_PER_W
    row0 = wid * ROWS_PER_W

    slots = ((idx0, word0, sem_out0, sem_idx0),
             (idx1, word1, sem_out1, sem_idx1))

    def out_slice(c):
        base = tok0 + c * CHUNK
        return out_hbm.at[pl.ds(base, CHUNK), pl.ds(0, D)]

    def do_chunk(t, s):
        idx_v, word_v, sem_out, sem_idx = slots[s]
        other_idx, _, _, other_sem = slots[1 - s]
        c = t * 2 + s
        pltpu.make_async_copy(
            x_hbm.at[pl.ds(row0, K)], idx_v, sem_idx).wait()

        @pl.when(c < N_CHUNKS - 1)
        def _():
            r = row0 + (c + 1) * K
            pltpu.async_copy(x_hbm.at[pl.ds(r, K)], other_idx, other_sem)

        @pl.when(t >= 1)
        def _():
            pltpu.make_async_copy(word_v, out_slice(c), sem_out).wait()

        cps = [pltpu.async_copy(
            w_hbm.at[idx_v.at[j]], word_v.at[pl.ds(j * 128, 128)], sem_g)
            for j in range(K)]
        for cp in cps:
            cp.wait()
        pltpu.async_copy(word_v, out_slice(c), sem_out)

    pltpu.async_copy(x_hbm.at[pl.ds(row0, K)], idx0, sem_idx0)

    def outer(t, carry):
        do_chunk(t, 0)
        do_chunk(t, 1)
        return carry

    lax.fori_loop(0, N_CHUNKS // 2, outer, 0)

    for s in range(2):
        idx_v, word_v, sem_out, _ = slots[s]
        pltpu.make_async_copy(word_v, out_slice(N_CHUNKS - 2 + s),
                              sem_out).wait()


def _tc_body(xe_ref, xn_ref, went_ref, wneg_ref, aliased_ref, out_ref,
             buf, sem):
    del aliased_ref
    e = xe_ref[...]
    n = xn_ref[...]
    went = went_ref[...]
    wneg = wneg_ref[...]
    ent = jnp.zeros((DE, TC_BLK), jnp.float32)
    neg = jnp.zeros((DE, TC_BLK), jnp.float32)
    for r in range(3):
        ent = ent + jnp.where(e == r, went[:, r:r + 1], 0.0)
        neg = neg + jnp.where(n == r, wneg[:, r:r + 1], 0.0)
    buf[...] = jnp.concatenate([ent, neg], axis=0).T
    i = pl.program_id(0)
    pltpu.make_async_copy(
        buf, out_ref.at[pl.ds(i * TC_BLK, TC_BLK), pl.ds(D, 2 * DE)],
        sem).start()
    pltpu.make_async_copy(
        buf, out_ref.at[pl.ds(i * TC_BLK, TC_BLK), pl.ds(D, 2 * DE)],
        sem).wait()


@jax.jit
def _run(x2d, xe2d, xn2d, w, w_ent, w_neg):
    mesh = plsc.VectorSubcoreMesh(core_axis_name="c", subcore_axis_name="s")
    sc = pl.kernel(
        _sc_body,
        out_type=jax.ShapeDtypeStruct((N_TOK, DO), jnp.float32),
        mesh=mesh,
        scratch_types=[
            pltpu.VMEM((K, 128), jnp.int32),
            pltpu.VMEM((CHUNK, D), jnp.float32),
            pltpu.VMEM((K, 128), jnp.int32),
            pltpu.VMEM((CHUNK, D), jnp.float32),
            pltpu.SemaphoreType.DMA,
            pltpu.SemaphoreType.DMA,
            pltpu.SemaphoreType.DMA,
            pltpu.SemaphoreType.DMA,
            pltpu.SemaphoreType.DMA,
        ],
        compiler_params=pltpu.CompilerParams(
            use_tc_tiling_on_sc=True, needs_layout_passes=False),
    )
    out_words = sc(x2d, w)

    grid = (N_TOK // TC_BLK,)
    return pl.pallas_call(
        _tc_body,
        grid=grid,
        in_specs=[
            pl.BlockSpec((1, TC_BLK), lambda i: (0, i)),
            pl.BlockSpec((1, TC_BLK), lambda i: (0, i)),
            pl.BlockSpec((DE, 3), lambda i: (0, 0)),
            pl.BlockSpec((DE, 3), lambda i: (0, 0)),
            pl.BlockSpec(memory_space=pl.ANY),
        ],
        out_specs=pl.BlockSpec(memory_space=pl.ANY),
        out_shape=jax.ShapeDtypeStruct((N_TOK, DO), jnp.float32),
        scratch_shapes=[pltpu.VMEM((TC_BLK, 2 * DE), jnp.float32),
                        pltpu.SemaphoreType.DMA],
        input_output_aliases={4: 0},
    )(xe2d, xn2d, w_ent, w_neg, out_words)


def kernel(x, x_entity, x_negation, target, text_inputs, use_elmo,
           W, W_entity, W_negation):
    out = _run(x.reshape(N_ROWS, 128).astype(jnp.int32),
               x_entity.reshape(1, N_TOK).astype(jnp.int32),
               x_negation.reshape(1, N_TOK).astype(jnp.int32),
               W, W_entity.T, W_negation.T)
    return out.reshape(B, L, DO)
